# baseline (speedup 1.0000x reference)
import functools

import jax
import jax.numpy as jnp
from jax import lax
from jax.experimental import pallas as pl
from jax.experimental.pallas import tpu as pltpu
from jax.experimental.pallas import tpu_sc as plsc

_B = 16384
_D = 64
_OUT_D = 76
_NW = 32
_BPW = _B // _NW
_CHUNK = 128
_NCH = _BPW // _CHUNK
_NBLK = _BPW // 16


def _body(uid_h, age_h, bw_h, bh_h, cal_h, sex_h, table_h, consts_h, sexf_h,
          out_h, uid_v, rows_v, feat_v, sex_v, consts_v, sexf_v, out_v, sem):
    c = lax.axis_index("c")
    s = lax.axis_index("s")
    wid = s * 2 + c
    base = wid * _BPW

    for j in range(_NCH):
        pltpu.sync_copy(uid_h.at[pl.ds(base + j * _CHUNK, _CHUNK)], uid_v.at[j])
    gathers = []
    for j in range(_NCH):
        cp = pltpu.make_async_copy(
            table_h.at[uid_v.at[j]], rows_v.at[pl.ds(j * _CHUNK, _CHUNK)], sem)
        cp.start()
        gathers.append(cp)

    pltpu.sync_copy(age_h.at[pl.ds(base, _BPW)], feat_v.at[0])
    pltpu.sync_copy(bw_h.at[pl.ds(base, _BPW)], feat_v.at[1])
    pltpu.sync_copy(bh_h.at[pl.ds(base, _BPW)], feat_v.at[2])
    pltpu.sync_copy(cal_h.at[pl.ds(base, _BPW)], feat_v.at[3])
    pltpu.sync_copy(sex_h.at[pl.ds(base, _BPW)], sex_v)
    pltpu.sync_copy(consts_h, consts_v)
    pltpu.sync_copy(sexf_h, sexf_v)

    lanes = lax.iota(jnp.int32, 16)

    def tail_blk(i, carry):
        obase = i * 16 * _OUT_D + lanes * _OUT_D
        for f in range(4):
            x = feat_v[f, pl.ds(i * 16, 16)]
            nv = (x - consts_v[f]) * consts_v[4 + f]
            plsc.store_scatter(out_v, [obase + (_D + f)], nv)
        sx = sex_v[pl.ds(i * 16, 16)] * 8
        for j in range(8):
            vals = plsc.load_gather(sexf_v, [sx + j])
            plsc.store_scatter(out_v, [obase + (_D + 4 + j)], vals)
        return carry

    lax.fori_loop(0, _NBLK, tail_blk, 0)

    for cp in gathers:
        cp.wait()

    def move_blk(i, carry):
        rvec = i * 16 + lanes
        obase = i * 16 * _OUT_D + lanes * _OUT_D
        for col in range(_D):
            cvec = jnp.full((16,), col, jnp.int32)
            v = plsc.load_gather(rows_v, [rvec, cvec])
            plsc.store_scatter(out_v, [obase + col], v)
        return carry

    lax.fori_loop(0, _NBLK, move_blk, 0)

    pltpu.sync_copy(out_v, out_h.at[pl.ds(base * _OUT_D, _BPW * _OUT_D)])


@jax.jit
def _run(uid, age, bw, bh, cal, sex, table, consts, sexf):
    mesh = plsc.VectorSubcoreMesh(core_axis_name="c", subcore_axis_name="s")
    f = functools.partial(
        pl.kernel,
        out_type=jax.ShapeDtypeStruct((_B * _OUT_D,), jnp.float32),
        mesh=mesh,
        compiler_params=pltpu.CompilerParams(
            needs_layout_passes=False, use_tc_tiling_on_sc=False),
        scratch_types=[
            pltpu.VMEM((_NCH, _CHUNK), jnp.int32),
            pltpu.VMEM((_BPW, _D), jnp.float32),
            pltpu.VMEM((4, _BPW), jnp.float32),
            pltpu.VMEM((_BPW,), jnp.int32),
            pltpu.VMEM((8, 16), jnp.float32),
            pltpu.VMEM((32,), jnp.float32),
            pltpu.VMEM((_BPW * _OUT_D,), jnp.float32),
            pltpu.SemaphoreType.DMA,
        ],
    )(_body)
    return f(uid, age, bw, bh, cal, sex, table, consts, sexf)


def kernel(User_ID, Age, Body_Weight, Body_Height, Cal_Need, sex, user_table,
           sex_table, feat_mean, feat_var):
    scale = 1.0 / jnp.maximum(jnp.sqrt(feat_var), 1e-7)
    consts = jnp.concatenate(
        [jnp.broadcast_to(feat_mean[:, None], (4, 16)),
         jnp.broadcast_to(scale[:, None], (4, 16))], axis=0)
    sexf = jnp.pad(sex_table.reshape(-1), (0, 8))
    flat = _run(User_ID.astype(jnp.int32), Age, Body_Weight, Body_Height,
                Cal_Need, sex.astype(jnp.int32), user_table, consts, sexf)
    return flat.reshape(_B, _OUT_D)


# speedup vs baseline: 1.3101x; 1.3101x over previous
import functools

import jax
import jax.numpy as jnp
from jax import lax
from jax.experimental import pallas as pl
from jax.experimental.pallas import tpu as pltpu
from jax.experimental.pallas import tpu_sc as plsc

_B = 16384
_D = 64
_OUT_D = 76
_NW = 32
_BPW = _B // _NW
_CHUNK = 128
_NCH = _BPW // _CHUNK
_NBLK = _BPW // 16


def _body(uid_h, age_h, bw_h, bh_h, cal_h, sex_h, table_h, consts_h, sexf_h,
          out_h, uid_v, rows_v, feat_v, sex_v, consts_v, sexf_v, tail_v, sem):
    c = lax.axis_index("c")
    s = lax.axis_index("s")
    wid = s * 2 + c
    base = wid * _BPW

    for j in range(_NCH):
        pltpu.sync_copy(uid_h.at[pl.ds(base + j * _CHUNK, _CHUNK)], uid_v.at[j])
    gathers = []
    for j in range(_NCH):
        cp = pltpu.make_async_copy(
            table_h.at[uid_v.at[j]], rows_v.at[pl.ds(j * _CHUNK, _CHUNK)], sem)
        cp.start()
        gathers.append(cp)

    pltpu.sync_copy(age_h.at[pl.ds(base, _BPW)], feat_v.at[0])
    pltpu.sync_copy(bw_h.at[pl.ds(base, _BPW)], feat_v.at[1])
    pltpu.sync_copy(bh_h.at[pl.ds(base, _BPW)], feat_v.at[2])
    pltpu.sync_copy(cal_h.at[pl.ds(base, _BPW)], feat_v.at[3])
    pltpu.sync_copy(sex_h.at[pl.ds(base, _BPW)], sex_v)
    pltpu.sync_copy(consts_h, consts_v)
    pltpu.sync_copy(sexf_h, sexf_v)

    lanes = lax.iota(jnp.int32, 16)

    def tail_blk(i, carry):
        rvec = i * 16 + lanes
        for f in range(4):
            x = feat_v[f, pl.ds(i * 16, 16)]
            nv = (x - consts_v[f]) * consts_v[4 + f]
            cvec = jnp.full((16,), f, jnp.int32)
            plsc.store_scatter(tail_v, [rvec, cvec], nv)
        sx = sex_v[pl.ds(i * 16, 16)] * 8
        for j in range(8):
            vals = plsc.load_gather(sexf_v, [sx + j])
            cvec = jnp.full((16,), 4 + j, jnp.int32)
            plsc.store_scatter(tail_v, [rvec, cvec], vals)
        return carry

    lax.fori_loop(0, _NBLK, tail_blk, 0)

    pltpu.sync_copy(tail_v, out_h.at[pl.ds(base, _BPW), pl.ds(_D, 12)])

    for cp in gathers:
        cp.wait()

    pltpu.sync_copy(rows_v, out_h.at[pl.ds(base, _BPW), pl.ds(0, _D)])


@jax.jit
def _run(uid, age, bw, bh, cal, sex, table, consts, sexf):
    mesh = plsc.VectorSubcoreMesh(core_axis_name="c", subcore_axis_name="s")
    f = functools.partial(
        pl.kernel,
        out_type=jax.ShapeDtypeStruct((_B, _OUT_D), jnp.float32),
        mesh=mesh,
        compiler_params=pltpu.CompilerParams(
            needs_layout_passes=False, use_tc_tiling_on_sc=False),
        scratch_types=[
            pltpu.VMEM((_NCH, _CHUNK), jnp.int32),
            pltpu.VMEM((_BPW, _D), jnp.float32),
            pltpu.VMEM((4, _BPW), jnp.float32),
            pltpu.VMEM((_BPW,), jnp.int32),
            pltpu.VMEM((8, 16), jnp.float32),
            pltpu.VMEM((32,), jnp.float32),
            pltpu.VMEM((_BPW, 12), jnp.float32),
            pltpu.SemaphoreType.DMA,
        ],
    )(_body)
    return f(uid, age, bw, bh, cal, sex, table, consts, sexf)


def kernel(User_ID, Age, Body_Weight, Body_Height, Cal_Need, sex, user_table,
           sex_table, feat_mean, feat_var):
    scale = 1.0 / jnp.maximum(jnp.sqrt(feat_var), 1e-7)
    consts = jnp.concatenate(
        [jnp.broadcast_to(feat_mean[:, None], (4, 16)),
         jnp.broadcast_to(scale[:, None], (4, 16))], axis=0)
    sexf = jnp.pad(sex_table.reshape(-1), (0, 8))
    return _run(User_ID.astype(jnp.int32), Age, Body_Weight, Body_Height,
                Cal_Need, sex.astype(jnp.int32), user_table, consts, sexf)


# speedup vs baseline: 2.0390x; 1.5564x over previous
import functools

import jax
import jax.numpy as jnp
from jax import lax
from jax.experimental import pallas as pl
from jax.experimental.pallas import tpu as pltpu
from jax.experimental.pallas import tpu_sc as plsc

_B = 16384
_D = 64
_OUT_D = 76
_V = 100001
_VMAIN = 99968
_VTAIL = _V - _VMAIN
_VPAD = _VMAIN + 128
_CHUNK = 4096
_NCHUNK = _B // _CHUNK


def _body(uid_h, feats_h, sex_h, tableT_h, tail_h, consts_h, sexf_h,
          outT_h, uid_v, colbuf_v, ocol_v, consts_v, sexf_v, sem):
    c = lax.axis_index("c")
    s = lax.axis_index("s")
    wid = s * 2 + c

    pltpu.sync_copy(consts_h, consts_v)
    pltpu.sync_copy(sexf_h, sexf_v)

    ncols = jnp.where(wid < 12, 3, 2)
    base_col = jnp.where(wid < 12, 3 * wid, 2 * wid + 12)

    def gather_chunk(k, col):
        def blk(i, carry):
            for u in range(4):
                off = i * 64 + u * 16
                idx = uid_v[pl.ds(off, 16)]
                vals = plsc.load_gather(colbuf_v, [idx])
                ocol_v[pl.ds(off, 16)] = vals
            return carry
        lax.fori_loop(0, _CHUNK // 64, blk, 0)
        pltpu.sync_copy(ocol_v, outT_h.at[col, pl.ds(k * _CHUNK, _CHUNK)])

    for slot in range(3):
        col = base_col + slot

        @pl.when(slot < ncols)
        def _process():
            @pl.when(col < _D)
            def _emb():
                pltpu.sync_copy(tableT_h.at[col, pl.ds(0, _VMAIN)],
                                colbuf_v.at[pl.ds(0, _VMAIN)])
                pltpu.sync_copy(tail_h.at[col],
                                colbuf_v.at[pl.ds(_VMAIN, 128)])
                for k in range(_NCHUNK):
                    pltpu.sync_copy(uid_h.at[pl.ds(k * _CHUNK, _CHUNK)], uid_v)
                    gather_chunk(k, col)

            @pl.when((col >= _D) & (col < _D + 4))
            def _feat():
                f = col - _D
                mean = consts_v[pl.ds(16 * f, 16)]
                scale = consts_v[pl.ds(_D + 16 * f, 16)]
                pltpu.sync_copy(feats_h.at[f], colbuf_v.at[pl.ds(0, _B)])
                for k in range(_NCHUNK):
                    def fblk(i, carry):
                        off = k * _CHUNK + i * 16
                        x = colbuf_v[pl.ds(off, 16)]
                        ocol_v[pl.ds(i * 16, 16)] = (x - mean) * scale
                        return carry
                    lax.fori_loop(0, _CHUNK // 16, fblk, 0)
                    pltpu.sync_copy(ocol_v,
                                    outT_h.at[col, pl.ds(k * _CHUNK, _CHUNK)])

            @pl.when(col >= _D + 4)
            def _sex():
                j = col - (_D + 4)
                for k in range(_NCHUNK):
                    pltpu.sync_copy(sex_h.at[pl.ds(k * _CHUNK, _CHUNK)], uid_v)
                    def sblk(i, carry):
                        off = i * 16
                        sv = uid_v[pl.ds(off, 16)]
                        vals = plsc.load_gather(sexf_v, [sv * 8 + j])
                        ocol_v[pl.ds(off, 16)] = vals
                        return carry
                    lax.fori_loop(0, _CHUNK // 16, sblk, 0)
                    pltpu.sync_copy(ocol_v,
                                    outT_h.at[col, pl.ds(k * _CHUNK, _CHUNK)])


@jax.jit
def _run(uid, feats, sex, tableT, tail, consts, sexf):
    mesh = plsc.VectorSubcoreMesh(core_axis_name="c", subcore_axis_name="s")
    f = functools.partial(
        pl.kernel,
        out_type=jax.ShapeDtypeStruct((_OUT_D, _B), jnp.float32),
        mesh=mesh,
        compiler_params=pltpu.CompilerParams(
            needs_layout_passes=False, use_tc_tiling_on_sc=True),
        scratch_types=[
            pltpu.VMEM((_CHUNK,), jnp.int32),
            pltpu.VMEM((_VPAD,), jnp.float32),
            pltpu.VMEM((_CHUNK,), jnp.float32),
            pltpu.VMEM((128,), jnp.float32),
            pltpu.VMEM((128,), jnp.float32),
            pltpu.SemaphoreType.DMA,
        ],
    )(_body)
    return f(uid, feats, sex, tableT, tail, consts, sexf)


def kernel(User_ID, Age, Body_Weight, Body_Height, Cal_Need, sex, user_table,
           sex_table, feat_mean, feat_var):
    scale = 1.0 / jnp.maximum(jnp.sqrt(feat_var), 1e-7)
    consts = jnp.concatenate(
        [jnp.broadcast_to(feat_mean[:, None], (4, 16)).reshape(-1),
         jnp.broadcast_to(scale[:, None], (4, 16)).reshape(-1)])
    feats = jnp.stack([Age, Body_Weight, Body_Height, Cal_Need])
    tableT = user_table.T
    tail = jnp.pad(user_table[_VMAIN:].T, ((0, 0), (0, 128 - _VTAIL)))
    sexf = jnp.pad(sex_table.reshape(-1), (0, 128 - 24))
    outT = _run(User_ID.astype(jnp.int32), feats, sex.astype(jnp.int32),
                tableT, tail, consts, sexf)
    return outT.T


# speedup vs baseline: 2.0954x; 1.0277x over previous
import functools

import jax
import jax.numpy as jnp
from jax import lax
from jax.experimental import pallas as pl
from jax.experimental.pallas import tpu as pltpu
from jax.experimental.pallas import tpu_sc as plsc

_B = 16384
_D = 64
_OUT_D = 76
_V = 100001
_VMAIN = 99968
_VTAIL = _V - _VMAIN
_VPAD = _VMAIN + 128
_CHUNK = 4096
_NCHUNK = _B // _CHUNK


def _body(uid_h, feats_h, sex_h, tableT_h, tail_h, consts_h, sexf_h,
          outT_h, uid_v, colbuf_v, ocol_v, consts_v, sexf_v, sem):
    c = lax.axis_index("c")
    s = lax.axis_index("s")
    wid = s * 2 + c

    pltpu.sync_copy(consts_h, consts_v)
    pltpu.sync_copy(sexf_h, sexf_v)
    pltpu.sync_copy(uid_h, uid_v)

    def gather_chunk(k, col):
        def blk(i, carry):
            for u in range(4):
                off = i * 64 + u * 16
                idx = uid_v[pl.ds(k * _CHUNK + off, 16)]
                vals = plsc.load_gather(colbuf_v, [idx])
                ocol_v[pl.ds(off, 16)] = vals
            return carry
        lax.fori_loop(0, _CHUNK // 64, blk, 0)
        pltpu.sync_copy(ocol_v, outT_h.at[col, pl.ds(k * _CHUNK, _CHUNK)])

    for slot in range(2):
        col = 2 * wid + slot
        pltpu.sync_copy(tableT_h.at[col, pl.ds(0, _VMAIN)],
                        colbuf_v.at[pl.ds(0, _VMAIN)])
        pltpu.sync_copy(tail_h.at[col], colbuf_v.at[pl.ds(_VMAIN, 128)])
        for k in range(_NCHUNK):
            gather_chunk(k, col)

    tcol = _D + wid

    @pl.when(wid < 4)
    def _feat():
        f = tcol - _D
        mean = consts_v[pl.ds(16 * wid, 16)]
        scale = consts_v[pl.ds(_D + 16 * wid, 16)]
        pltpu.sync_copy(feats_h.at[f], colbuf_v.at[pl.ds(0, _B)])
        for k in range(_NCHUNK):
            def fblk(i, carry):
                off = k * _CHUNK + i * 16
                x = colbuf_v[pl.ds(off, 16)]
                ocol_v[pl.ds(i * 16, 16)] = (x - mean) * scale
                return carry
            lax.fori_loop(0, _CHUNK // 16, fblk, 0)
            pltpu.sync_copy(ocol_v, outT_h.at[tcol, pl.ds(k * _CHUNK, _CHUNK)])

    @pl.when((wid >= 4) & (wid < 12))
    def _sex():
        j = tcol - (_D + 4)
        pltpu.sync_copy(sex_h, uid_v)
        for k in range(_NCHUNK):
            def sblk(i, carry):
                off = i * 16
                sv = uid_v[pl.ds(k * _CHUNK + off, 16)]
                vals = plsc.load_gather(sexf_v, [sv * 8 + j])
                ocol_v[pl.ds(off, 16)] = vals
                return carry
            lax.fori_loop(0, _CHUNK // 16, sblk, 0)
            pltpu.sync_copy(ocol_v, outT_h.at[tcol, pl.ds(k * _CHUNK, _CHUNK)])


@jax.jit
def _run(uid, feats, sex, tableT, tail, consts, sexf):
    mesh = plsc.VectorSubcoreMesh(core_axis_name="c", subcore_axis_name="s")
    f = functools.partial(
        pl.kernel,
        out_type=jax.ShapeDtypeStruct((_OUT_D, _B), jnp.float32),
        mesh=mesh,
        compiler_params=pltpu.CompilerParams(
            needs_layout_passes=False, use_tc_tiling_on_sc=True),
        scratch_types=[
            pltpu.VMEM((_B,), jnp.int32),
            pltpu.VMEM((_VPAD,), jnp.float32),
            pltpu.VMEM((_CHUNK,), jnp.float32),
            pltpu.VMEM((128,), jnp.float32),
            pltpu.VMEM((128,), jnp.float32),
            pltpu.SemaphoreType.DMA,
        ],
    )(_body)
    return f(uid, feats, sex, tableT, tail, consts, sexf)


def kernel(User_ID, Age, Body_Weight, Body_Height, Cal_Need, sex, user_table,
           sex_table, feat_mean, feat_var):
    scale = 1.0 / jnp.maximum(jnp.sqrt(feat_var), 1e-7)
    consts = jnp.concatenate(
        [jnp.broadcast_to(feat_mean[:, None], (4, 16)).reshape(-1),
         jnp.broadcast_to(scale[:, None], (4, 16)).reshape(-1)])
    feats = jnp.stack([Age, Body_Weight, Body_Height, Cal_Need])
    tableT = user_table.T
    tail = jnp.pad(user_table[_VMAIN:].T, ((0, 0), (0, 128 - _VTAIL)))
    sexf = jnp.pad(sex_table.reshape(-1), (0, 128 - 24))
    outT = _run(User_ID.astype(jnp.int32), feats, sex.astype(jnp.int32),
                tableT, tail, consts, sexf)
    return outT.T


# speedup vs baseline: 2.5957x; 1.2388x over previous
import functools

import jax
import jax.numpy as jnp
from jax import lax
from jax.experimental import pallas as pl
from jax.experimental.pallas import tpu as pltpu
from jax.experimental.pallas import tpu_sc as plsc

_B = 16384
_D = 64
_OUT_D = 76
_V = 100001
_VMAIN = 99968
_VTAIL = _V - _VMAIN
_VPAD = _VMAIN + 128
_CHUNK = 4096
_NCHUNK = _B // _CHUNK


def _body(uid_h, feats_h, sex_h, tableT_h, tail_h, consts_h, sexf_h,
          outT_h, uid_v, colbuf_v, ocol0_v, ocol1_v, consts_v, sexf_v,
          sem, osem0, osem1):
    c = lax.axis_index("c")
    s = lax.axis_index("s")
    wid = s * 2 + c

    pltpu.sync_copy(consts_h, consts_v)
    pltpu.sync_copy(sexf_h, sexf_v)
    pltpu.sync_copy(uid_h, uid_v)

    ocols = (ocol0_v, ocol1_v)
    osems = (osem0, osem1)
    pending = [None, None]

    def out_write(k, col):
        buf = ocols[k % 2]
        cp = pltpu.make_async_copy(
            buf, outT_h.at[col, pl.ds(k * _CHUNK, _CHUNK)], osems[k % 2])
        cp.start()
        pending[k % 2] = cp

    def drain(k):
        if pending[k % 2] is not None:
            pending[k % 2].wait()
            pending[k % 2] = None

    def gather_chunk(k, col):
        drain(k)
        buf = ocols[k % 2]

        def blk(i, carry):
            for u in range(8):
                off = i * 128 + u * 16
                idx = uid_v[pl.ds(k * _CHUNK + off, 16)]
                vals = plsc.load_gather(colbuf_v, [idx])
                buf[pl.ds(off, 16)] = vals
            return carry
        lax.fori_loop(0, _CHUNK // 128, blk, 0)
        out_write(k, col)

    for slot in range(2):
        col = 2 * wid + slot
        pltpu.sync_copy(tableT_h.at[col, pl.ds(0, _VMAIN)],
                        colbuf_v.at[pl.ds(0, _VMAIN)])
        pltpu.sync_copy(tail_h.at[col], colbuf_v.at[pl.ds(_VMAIN, 128)])
        for k in range(_NCHUNK):
            gather_chunk(k, col)

    drain(0)
    drain(1)

    tcol = _D + wid

    @pl.when(wid < 4)
    def _feat():
        mean = consts_v[pl.ds(16 * wid, 16)]
        scale = consts_v[pl.ds(_D + 16 * wid, 16)]
        pltpu.sync_copy(feats_h.at[wid], colbuf_v.at[pl.ds(0, _B)])
        for k in range(_NCHUNK):
            def fblk(i, carry):
                for u in range(8):
                    off = i * 128 + u * 16
                    x = colbuf_v[pl.ds(k * _CHUNK + off, 16)]
                    ocol0_v[pl.ds(off, 16)] = (x - mean) * scale
                return carry
            lax.fori_loop(0, _CHUNK // 128, fblk, 0)
            pltpu.sync_copy(ocol0_v, outT_h.at[tcol, pl.ds(k * _CHUNK, _CHUNK)])

    @pl.when((wid >= 4) & (wid < 12))
    def _sex():
        j = tcol - (_D + 4)
        pltpu.sync_copy(sex_h, uid_v)
        for k in range(_NCHUNK):
            def sblk(i, carry):
                for u in range(8):
                    off = i * 128 + u * 16
                    sv = uid_v[pl.ds(k * _CHUNK + off, 16)]
                    vals = plsc.load_gather(sexf_v, [sv * 8 + j])
                    ocol0_v[pl.ds(off, 16)] = vals
                return carry
            lax.fori_loop(0, _CHUNK // 128, sblk, 0)
            pltpu.sync_copy(ocol0_v, outT_h.at[tcol, pl.ds(k * _CHUNK, _CHUNK)])


@jax.jit
def _run(uid, feats, sex, tableT, tail, consts, sexf):
    mesh = plsc.VectorSubcoreMesh(core_axis_name="c", subcore_axis_name="s")
    f = functools.partial(
        pl.kernel,
        out_type=jax.ShapeDtypeStruct((_OUT_D, _B), jnp.float32),
        mesh=mesh,
        compiler_params=pltpu.CompilerParams(
            needs_layout_passes=False, use_tc_tiling_on_sc=True),
        scratch_types=[
            pltpu.VMEM((_B,), jnp.int32),
            pltpu.VMEM((_VPAD,), jnp.float32),
            pltpu.VMEM((_CHUNK,), jnp.float32),
            pltpu.VMEM((_CHUNK,), jnp.float32),
            pltpu.VMEM((128,), jnp.float32),
            pltpu.VMEM((128,), jnp.float32),
            pltpu.SemaphoreType.DMA,
            pltpu.SemaphoreType.DMA,
            pltpu.SemaphoreType.DMA,
        ],
    )(_body)
    return f(uid, feats, sex, tableT, tail, consts, sexf)


def kernel(User_ID, Age, Body_Weight, Body_Height, Cal_Need, sex, user_table,
           sex_table, feat_mean, feat_var):
    scale = 1.0 / jnp.maximum(jnp.sqrt(feat_var), 1e-7)
    consts = jnp.concatenate(
        [jnp.broadcast_to(feat_mean[:, None], (4, 16)).reshape(-1),
         jnp.broadcast_to(scale[:, None], (4, 16)).reshape(-1)])
    feats = jnp.stack([Age, Body_Weight, Body_Height, Cal_Need])
    tableT = user_table.T
    tail = jnp.pad(user_table[_VMAIN:].T, ((0, 0), (0, 128 - _VTAIL)))
    sexf = jnp.pad(sex_table.reshape(-1), (0, 128 - 24))
    outT = _run(User_ID.astype(jnp.int32), feats, sex.astype(jnp.int32),
                tableT, tail, consts, sexf)
    return outT.T


# speedup vs baseline: 2.6347x; 1.0150x over previous
import functools

import jax
import jax.numpy as jnp
from jax import lax
from jax.experimental import pallas as pl
from jax.experimental.pallas import tpu as pltpu
from jax.experimental.pallas import tpu_sc as plsc

_B = 16384
_D = 64
_OUT_D = 76
_V = 100001
_VMAIN = 99968
_VTAIL = _V - _VMAIN
_VPAD = _VMAIN + 128
_CHUNK = 4096
_NCHUNK = _B // _CHUNK


def _body(uid_h, feats_h, sex_h, tableT_h, tail_h, consts_h, sexf_h,
          outT_h, uid_v, colbuf_v, ocol0_v, ocol1_v, consts_v, sexf_v,
          sem, osem0, osem1):
    c = lax.axis_index("c")
    s = lax.axis_index("s")
    wid = s * 2 + c

    pltpu.sync_copy(consts_h, consts_v)
    pltpu.sync_copy(sexf_h, sexf_v)
    pltpu.sync_copy(uid_h, uid_v)

    ocols = (ocol0_v, ocol1_v)
    osems = (osem0, osem1)
    pending = [None, None]

    def out_write(k, col):
        buf = ocols[k % 2]
        cp = pltpu.make_async_copy(
            buf, outT_h.at[col, pl.ds(k * _CHUNK, _CHUNK)], osems[k % 2])
        cp.start()
        pending[k % 2] = cp

    def drain(k):
        if pending[k % 2] is not None:
            pending[k % 2].wait()
            pending[k % 2] = None

    def gather_chunk(k, col):
        drain(k)
        buf = ocols[k % 2]

        def blk(i, carry):
            for u in range(8):
                off = i * 128 + u * 16
                idx = uid_v[pl.ds(k * _CHUNK + off, 16)]
                vals = plsc.load_gather(colbuf_v, [idx])
                buf[pl.ds(off, 16)] = vals
            return carry
        lax.fori_loop(0, _CHUNK // 128, blk, 0)
        out_write(k, col)

    _PARTS = (0, 25088, 50176, 75264, _VMAIN)
    for slot in range(2):
        col = 2 * wid + slot
        cps = []
        for p in range(4):
            lo, hi = _PARTS[p], _PARTS[p + 1]
            cp = pltpu.make_async_copy(
                tableT_h.at[col, pl.ds(lo, hi - lo)],
                colbuf_v.at[pl.ds(lo, hi - lo)], sem)
            cp.start()
            cps.append(cp)
        cp = pltpu.make_async_copy(
            tail_h.at[col], colbuf_v.at[pl.ds(_VMAIN, 128)], sem)
        cp.start()
        cps.append(cp)
        for cp in cps:
            cp.wait()
        for k in range(_NCHUNK):
            gather_chunk(k, col)

    drain(0)
    drain(1)

    tcol = _D + wid

    @pl.when(wid < 4)
    def _feat():
        mean = consts_v[pl.ds(16 * wid, 16)]
        scale = consts_v[pl.ds(_D + 16 * wid, 16)]
        pltpu.sync_copy(feats_h.at[wid], colbuf_v.at[pl.ds(0, _B)])
        for k in range(_NCHUNK):
            def fblk(i, carry):
                for u in range(8):
                    off = i * 128 + u * 16
                    x = colbuf_v[pl.ds(k * _CHUNK + off, 16)]
                    ocol0_v[pl.ds(off, 16)] = (x - mean) * scale
                return carry
            lax.fori_loop(0, _CHUNK // 128, fblk, 0)
            pltpu.sync_copy(ocol0_v, outT_h.at[tcol, pl.ds(k * _CHUNK, _CHUNK)])

    @pl.when((wid >= 4) & (wid < 12))
    def _sex():
        j = tcol - (_D + 4)
        pltpu.sync_copy(sex_h, uid_v)
        for k in range(_NCHUNK):
            def sblk(i, carry):
                for u in range(8):
                    off = i * 128 + u * 16
                    sv = uid_v[pl.ds(k * _CHUNK + off, 16)]
                    vals = plsc.load_gather(sexf_v, [sv * 8 + j])
                    ocol0_v[pl.ds(off, 16)] = vals
                return carry
            lax.fori_loop(0, _CHUNK // 128, sblk, 0)
            pltpu.sync_copy(ocol0_v, outT_h.at[tcol, pl.ds(k * _CHUNK, _CHUNK)])


@jax.jit
def _run(uid, feats, sex, tableT, tail, consts, sexf):
    mesh = plsc.VectorSubcoreMesh(core_axis_name="c", subcore_axis_name="s")
    f = functools.partial(
        pl.kernel,
        out_type=jax.ShapeDtypeStruct((_OUT_D, _B), jnp.float32),
        mesh=mesh,
        compiler_params=pltpu.CompilerParams(
            needs_layout_passes=False, use_tc_tiling_on_sc=True),
        scratch_types=[
            pltpu.VMEM((_B,), jnp.int32),
            pltpu.VMEM((_VPAD,), jnp.float32),
            pltpu.VMEM((_CHUNK,), jnp.float32),
            pltpu.VMEM((_CHUNK,), jnp.float32),
            pltpu.VMEM((128,), jnp.float32),
            pltpu.VMEM((128,), jnp.float32),
            pltpu.SemaphoreType.DMA,
            pltpu.SemaphoreType.DMA,
            pltpu.SemaphoreType.DMA,
        ],
    )(_body)
    return f(uid, feats, sex, tableT, tail, consts, sexf)


def kernel(User_ID, Age, Body_Weight, Body_Height, Cal_Need, sex, user_table,
           sex_table, feat_mean, feat_var):
    scale = 1.0 / jnp.maximum(jnp.sqrt(feat_var), 1e-7)
    consts = jnp.concatenate(
        [jnp.broadcast_to(feat_mean[:, None], (4, 16)).reshape(-1),
         jnp.broadcast_to(scale[:, None], (4, 16)).reshape(-1)])
    feats = jnp.stack([Age, Body_Weight, Body_Height, Cal_Need])
    tableT = user_table.T
    tail = jnp.pad(user_table[_VMAIN:].T, ((0, 0), (0, 128 - _VTAIL)))
    sexf = jnp.pad(sex_table.reshape(-1), (0, 128 - 24))
    outT = _run(User_ID.astype(jnp.int32), feats, sex.astype(jnp.int32),
                tableT, tail, consts, sexf)
    return outT.T
